# baseline (speedup 1.0000x reference)
import functools

import jax
import jax.numpy as jnp
from jax import lax
from jax.experimental import pallas as pl
from jax.experimental.pallas import tpu as pltpu
from jax.experimental.pallas import tpu_sc as plsc

_K = 16
_F = 64
_L = 16
_NC = 2
_NS = 16
_NW = _NC * _NS
_C = 32
_CK = _C * _K


def _mm_relu(x, w, b, out_rows, block_rows=512):
    v, d = x.shape
    f = w.shape[1]
    assert out_rows % block_rows == 0
    grid = out_rows // block_rows
    last_blk = (v + block_rows - 1) // block_rows - 1

    def body(x_ref, w_ref, b_ref, o_ref):
        acc = jnp.dot(x_ref[...], w_ref[...], preferred_element_type=jnp.float32)
        o_ref[...] = jnp.maximum(acc + b_ref[...], 0.0)

    return pl.pallas_call(
        body,
        grid=(grid,),
        in_specs=[
            pl.BlockSpec((block_rows, d), lambda i: (jnp.minimum(i, last_blk), 0)),
            pl.BlockSpec((d, f), lambda i: (0, 0)),
            pl.BlockSpec((1, f), lambda i: (0, 0)),
        ],
        out_specs=pl.BlockSpec((block_rows, f), lambda i: (i, 0)),
        out_shape=jax.ShapeDtypeStruct((out_rows, f), jnp.float32),
    )(x, w, b.reshape(1, f))


@functools.lru_cache(maxsize=None)
def _make_knn(vp: int):
    n_w = vp // _NW
    n_chunks = n_w // _C
    n_pairs = n_chunks // 2
    assert n_pairs * 2 * _C == n_w
    mesh = plsc.VectorSubcoreMesh(core_axis_name="c", subcore_axis_name="s")

    @functools.partial(
        pl.kernel,
        out_type=jax.ShapeDtypeStruct((vp, 2 * _F), jnp.float32),
        mesh=mesh,
        compiler_params=pltpu.CompilerParams(use_tc_tiling_on_sc=False),
        scratch_types=[
            pltpu.VMEM((2, _CK), jnp.int32),
            pltpu.VMEM((2, _CK), jnp.float32),
            pltpu.VMEM((2, _CK, _F), jnp.float32),
            pltpu.VMEM((2, _C, _F), jnp.float32),
            pltpu.VMEM((2, _C, 2 * _F), jnp.float32),
            pltpu.SemaphoreType.DMA,
            pltpu.SemaphoreType.DMA,
            pltpu.SemaphoreType.DMA,
            pltpu.SemaphoreType.DMA,
        ],
    )
    def knn(feat_hbm, idx_hbm, dsq_hbm, out_hbm,
            idx_v, dsq_v, rows_v, self_v, out_v,
            sem_in0, sem_in1, sem_g0, sem_g1):
        wid = lax.axis_index("s") * _NC + lax.axis_index("c")
        w_base = wid * n_w
        sem_in = (sem_in0, sem_in1)
        sem_g = (sem_g0, sem_g1)

        def start_in(par, ci):
            base = w_base + ci * _C
            pltpu.async_copy(idx_hbm.at[pl.ds(base * _K, _CK)], idx_v.at[par], sem_in[par])
            pltpu.async_copy(dsq_hbm.at[pl.ds(base * _K, _CK)], dsq_v.at[par], sem_in[par])
            pltpu.async_copy(feat_hbm.at[pl.ds(base, _C)], self_v.at[par], sem_in[par])

        def wait_in(par):
            pltpu.make_async_copy(idx_hbm.at[pl.ds(0, _CK)], idx_v.at[par], sem_in[par]).wait()
            pltpu.make_async_copy(dsq_hbm.at[pl.ds(0, _CK)], dsq_v.at[par], sem_in[par]).wait()
            pltpu.make_async_copy(feat_hbm.at[pl.ds(0, _C)], self_v.at[par], sem_in[par]).wait()

        def start_gather(par):
            pltpu.async_copy(feat_hbm.at[idx_v.at[par]], rows_v.at[par], sem_g[par])

        def wait_gather(par):
            pltpu.make_async_copy(feat_hbm.at[pl.ds(0, _CK)], rows_v.at[par], sem_g[par]).wait()

        def compute(par, ci):
            base = w_base + ci * _C

            def vert_body(v, c2):
                wv = jnp.exp(dsq_v[par, pl.ds(v * _K, _K)] * -10.0)
                r0 = v * _K
                w0 = wv[0]
                acc_m = []
                acc_x = []
                for fb in range(_F // _L):
                    g = rows_v[par, r0, pl.ds(fb * _L, _L)] * w0
                    acc_m.append(g)
                    acc_x.append(g)
                for k in range(1, _K):
                    wk = wv[k]
                    for fb in range(_F // _L):
                        g = rows_v[par, r0 + k, pl.ds(fb * _L, _L)] * wk
                        acc_m[fb] = acc_m[fb] + g
                        acc_x[fb] = jnp.maximum(acc_x[fb], g)
                for fb in range(_F // _L):
                    s = self_v[par, v, pl.ds(fb * _L, _L)]
                    out_v[par, v, pl.ds(fb * _L, _L)] = acc_m[fb] * (1.0 / _K) - s
                    out_v[par, v, pl.ds(_F + fb * _L, _L)] = acc_x[fb] - s
                return c2

            lax.fori_loop(0, _C, vert_body, 0)
            pltpu.sync_copy(out_v.at[par], out_hbm.at[pl.ds(base, _C)])

        start_in(0, 0)
        start_in(1, 1)
        wait_in(0)
        start_gather(0)

        last = n_chunks - 1

        def pair_body(i, carry):
            c0 = 2 * i
            with jax.named_scope("wg0"):
                wait_gather(0)
            with jax.named_scope("wi1"):
                wait_in(1)
                start_gather(1)
            with jax.named_scope("comp0"):
                compute(0, c0)
            with jax.named_scope("si0"):
                start_in(0, jnp.minimum(c0 + 2, last))
            with jax.named_scope("wg1"):
                wait_gather(1)
            with jax.named_scope("wi0"):
                wait_in(0)
                start_gather(0)
            with jax.named_scope("comp1"):
                compute(1, c0 + 1)
            with jax.named_scope("si1"):
                start_in(1, jnp.minimum(c0 + 3, last))
            return carry

        lax.fori_loop(0, n_pairs, pair_body, 0)
        wait_gather(0)
        wait_in(1)

    return knn


def kernel(x, neighbor_indices, distancesq, W1, b1, W2, b2):
    v, d = x.shape
    span = _NW * _C * 2
    vp = ((v + span - 1) // span) * span
    pad = vp - v
    xp = jnp.pad(x, ((0, pad), (0, 0)))
    idx_flat = jnp.pad(neighbor_indices, ((0, pad), (0, 0))).reshape(-1)
    dsq_flat = jnp.pad(distancesq, ((0, pad), (0, 0))).reshape(-1)

    knn = _make_knn(vp)
    h1 = _mm_relu(xp, W1, b1, vp)
    out1 = knn(h1, idx_flat, dsq_flat)
    h2 = _mm_relu(out1, W2, b2, vp)
    out2 = knn(h2, idx_flat, dsq_flat)
    return jnp.concatenate([out1[:v], out2[:v], x], axis=-1)


# speedup vs baseline: 1.0030x; 1.0030x over previous
import functools

import jax
import jax.numpy as jnp
from jax import lax
from jax.experimental import pallas as pl
from jax.experimental.pallas import tpu as pltpu
from jax.experimental.pallas import tpu_sc as plsc

_K = 16
_F = 64
_L = 16
_NC = 2
_NS = 16
_NW = _NC * _NS
_C = 32
_CK = _C * _K


def _mm_relu(x, w, b, out_rows, block_rows=512):
    v, d = x.shape
    f = w.shape[1]
    assert out_rows % block_rows == 0
    grid = out_rows // block_rows
    last_blk = (v + block_rows - 1) // block_rows - 1

    def body(x_ref, w_ref, b_ref, o_ref):
        acc = jnp.dot(x_ref[...], w_ref[...], preferred_element_type=jnp.float32)
        o_ref[...] = jnp.maximum(acc + b_ref[...], 0.0)

    return pl.pallas_call(
        body,
        grid=(grid,),
        in_specs=[
            pl.BlockSpec((block_rows, d), lambda i: (jnp.minimum(i, last_blk), 0)),
            pl.BlockSpec((d, f), lambda i: (0, 0)),
            pl.BlockSpec((1, f), lambda i: (0, 0)),
        ],
        out_specs=pl.BlockSpec((block_rows, f), lambda i: (i, 0)),
        out_shape=jax.ShapeDtypeStruct((out_rows, f), jnp.float32),
    )(x, w, b.reshape(1, f))


@functools.lru_cache(maxsize=None)
def _make_knn(vp: int):
    n_w = vp // _NW
    n_chunks = n_w // _C
    n_pairs = n_chunks // 2
    assert n_pairs * 2 * _C == n_w
    mesh = plsc.VectorSubcoreMesh(core_axis_name="c", subcore_axis_name="s")

    @functools.partial(
        pl.kernel,
        out_type=jax.ShapeDtypeStruct((vp, 2 * _F), jnp.float32),
        mesh=mesh,
        compiler_params=pltpu.CompilerParams(use_tc_tiling_on_sc=False),
        scratch_types=[
            pltpu.VMEM((2, _CK), jnp.int32),
            pltpu.VMEM((2, _CK), jnp.float32),
            pltpu.VMEM((2, _CK, _F), jnp.float32),
            pltpu.VMEM((2, _C, _F), jnp.float32),
            pltpu.VMEM((2, _C, 2 * _F), jnp.float32),
            pltpu.SemaphoreType.DMA,
            pltpu.SemaphoreType.DMA,
            pltpu.SemaphoreType.DMA,
            pltpu.SemaphoreType.DMA,
        ],
    )
    def knn(feat_hbm, idx_hbm, dsq_hbm, out_hbm,
            idx_v, dsq_v, rows_v, self_v, out_v,
            sem_in0, sem_in1, sem_g0, sem_g1):
        wid = lax.axis_index("s") * _NC + lax.axis_index("c")
        w_base = wid * n_w
        sem_in = (sem_in0, sem_in1)
        sem_g = (sem_g0, sem_g1)

        def start_in(par, ci):
            base = w_base + ci * _C
            pltpu.async_copy(idx_hbm.at[pl.ds(base * _K, _CK)], idx_v.at[par], sem_in[par])
            pltpu.async_copy(dsq_hbm.at[pl.ds(base * _K, _CK)], dsq_v.at[par], sem_in[par])
            pltpu.async_copy(feat_hbm.at[pl.ds(base, _C)], self_v.at[par], sem_in[par])

        def wait_in(par):
            pltpu.make_async_copy(idx_hbm.at[pl.ds(0, _CK)], idx_v.at[par], sem_in[par]).wait()
            pltpu.make_async_copy(dsq_hbm.at[pl.ds(0, _CK)], dsq_v.at[par], sem_in[par]).wait()
            pltpu.make_async_copy(feat_hbm.at[pl.ds(0, _C)], self_v.at[par], sem_in[par]).wait()

        def start_gather(par):
            pltpu.async_copy(feat_hbm.at[idx_v.at[par]], rows_v.at[par], sem_g[par])

        def wait_gather(par):
            pltpu.make_async_copy(feat_hbm.at[pl.ds(0, _CK)], rows_v.at[par], sem_g[par]).wait()

        def compute(par, ci):
            base = w_base + ci * _C

            def vert_body(v, c2):
                wv = jnp.exp(dsq_v[par, pl.ds(v * _K, _K)] * -10.0)
                r0 = v * _K
                w0 = wv[0]
                acc_m = []
                acc_x = []
                for fb in range(_F // _L):
                    g = rows_v[par, r0, pl.ds(fb * _L, _L)] * w0
                    acc_m.append(g)
                    acc_x.append(g)
                for k in range(1, _K):
                    wk = wv[k]
                    for fb in range(_F // _L):
                        g = rows_v[par, r0 + k, pl.ds(fb * _L, _L)] * wk
                        acc_m[fb] = acc_m[fb] + g
                        acc_x[fb] = jnp.maximum(acc_x[fb], g)
                for fb in range(_F // _L):
                    s = self_v[par, v, pl.ds(fb * _L, _L)]
                    out_v[par, v, pl.ds(fb * _L, _L)] = acc_m[fb] * (1.0 / _K) - s
                    out_v[par, v, pl.ds(_F + fb * _L, _L)] = acc_x[fb] - s
                return c2

            lax.fori_loop(0, _C, vert_body, 0)
            pltpu.sync_copy(out_v.at[par], out_hbm.at[pl.ds(base, _C)])

        start_in(0, 0)
        start_in(1, 1)
        wait_in(0)
        start_gather(0)

        def pair_body(i, carry):
            c0 = 2 * i
            wait_gather(0)
            wait_in(1)
            start_gather(1)
            compute(0, c0)
            start_in(0, c0 + 2)
            wait_gather(1)
            wait_in(0)
            start_gather(0)
            compute(1, c0 + 1)
            start_in(1, c0 + 3)
            return carry

        lax.fori_loop(0, n_pairs - 1, pair_body, 0)
        c0 = n_chunks - 2
        wait_gather(0)
        wait_in(1)
        start_gather(1)
        compute(0, c0)
        wait_gather(1)
        compute(1, c0 + 1)

    return knn


def kernel(x, neighbor_indices, distancesq, W1, b1, W2, b2):
    v, d = x.shape
    span = _NW * _C * 2
    vp = ((v + span - 1) // span) * span
    pad = vp - v
    xp = jnp.pad(x, ((0, pad), (0, 0)))
    idx_flat = jnp.pad(neighbor_indices, ((0, pad), (0, 0))).reshape(-1)
    dsq_flat = jnp.pad(distancesq, ((0, pad), (0, 0))).reshape(-1)

    knn = _make_knn(vp)
    h1 = _mm_relu(xp, W1, b1, vp)
    out1 = knn(h1, idx_flat, dsq_flat)
    h2 = _mm_relu(out1, W2, b2, vp)
    out2 = knn(h2, idx_flat, dsq_flat)
    return jnp.concatenate([out1[:v], out2[:v], x], axis=-1)


# speedup vs baseline: 1.0041x; 1.0011x over previous
import functools

import jax
import jax.numpy as jnp
from jax import lax
from jax.experimental import pallas as pl
from jax.experimental.pallas import tpu as pltpu
from jax.experimental.pallas import tpu_sc as plsc

_K = 16
_F = 64
_L = 16
_NC = 2
_NS = 16
_NW = _NC * _NS
_C = 32
_CK = _C * _K


def _mm_relu(x, w, b, out_rows, block_rows=512):
    v, d = x.shape
    f = w.shape[1]
    assert out_rows % block_rows == 0
    grid = out_rows // block_rows
    last_blk = (v + block_rows - 1) // block_rows - 1

    def body(x_ref, w_ref, b_ref, o_ref):
        acc = jnp.dot(x_ref[...], w_ref[...], preferred_element_type=jnp.float32)
        o_ref[...] = jnp.maximum(acc + b_ref[...], 0.0)

    return pl.pallas_call(
        body,
        grid=(grid,),
        in_specs=[
            pl.BlockSpec((block_rows, d), lambda i: (jnp.minimum(i, last_blk), 0)),
            pl.BlockSpec((d, f), lambda i: (0, 0)),
            pl.BlockSpec((1, f), lambda i: (0, 0)),
        ],
        out_specs=pl.BlockSpec((block_rows, f), lambda i: (i, 0)),
        out_shape=jax.ShapeDtypeStruct((out_rows, f), jnp.float32),
    )(x, w, b.reshape(1, f))


@functools.lru_cache(maxsize=None)
def _make_knn(vp: int):
    n_w = vp // _NW
    n_chunks = n_w // _C
    n_pairs = n_chunks // 2
    assert n_pairs * 2 * _C == n_w
    mesh = plsc.VectorSubcoreMesh(core_axis_name="c", subcore_axis_name="s")

    @functools.partial(
        pl.kernel,
        out_type=jax.ShapeDtypeStruct((vp, 2 * _F), jnp.float32),
        mesh=mesh,
        compiler_params=pltpu.CompilerParams(use_tc_tiling_on_sc=False),
        scratch_types=[
            pltpu.VMEM((2, _CK), jnp.int32),
            pltpu.VMEM((2, _CK), jnp.float32),
            pltpu.VMEM((2, _CK, _F), jnp.float32),
            pltpu.VMEM((2, _C, _F), jnp.float32),
            pltpu.VMEM((2, _C, 2 * _F), jnp.float32),
            pltpu.SemaphoreType.DMA,
            pltpu.SemaphoreType.DMA,
            pltpu.SemaphoreType.DMA,
            pltpu.SemaphoreType.DMA,
        ],
    )
    def knn(feat_hbm, idx_hbm, dsq_hbm, out_hbm,
            idx_v, dsq_v, rows_v, self_v, out_v,
            sem_in0, sem_in1, sem_g0, sem_g1):
        wid = lax.axis_index("s") * _NC + lax.axis_index("c")
        w_base = wid * n_w
        sem_in = (sem_in0, sem_in1)
        sem_g = (sem_g0, sem_g1)

        def start_in(par, ci):
            base = w_base + ci * _C
            pltpu.async_copy(idx_hbm.at[pl.ds(base * _K, _CK)], idx_v.at[par], sem_in[par])
            pltpu.async_copy(dsq_hbm.at[pl.ds(base * _K, _CK)], dsq_v.at[par], sem_in[par])
            pltpu.async_copy(feat_hbm.at[pl.ds(base, _C)], self_v.at[par], sem_in[par])

        def wait_in(par):
            pltpu.make_async_copy(idx_hbm.at[pl.ds(0, _CK)], idx_v.at[par], sem_in[par]).wait()
            pltpu.make_async_copy(dsq_hbm.at[pl.ds(0, _CK)], dsq_v.at[par], sem_in[par]).wait()
            pltpu.make_async_copy(feat_hbm.at[pl.ds(0, _C)], self_v.at[par], sem_in[par]).wait()

        def start_gather(par):
            pltpu.async_copy(feat_hbm.at[idx_v.at[par]], rows_v.at[par], sem_g[par])

        def wait_gather(par):
            pltpu.make_async_copy(feat_hbm.at[pl.ds(0, _CK)], rows_v.at[par], sem_g[par]).wait()

        def compute(par, ci):
            base = w_base + ci * _C

            def vert_body(v, c2):
                wv = jnp.exp(dsq_v[par, pl.ds(v * _K, _K)] * -10.0)
                r0 = v * _K
                w0 = wv[0]
                acc_m = []
                acc_x = []
                for fb in range(_F // _L):
                    g = rows_v[par, r0, pl.ds(fb * _L, _L)] * w0
                    acc_m.append(g)
                    acc_x.append(g)
                for k in range(1, _K):
                    wk = wv[k]
                    for fb in range(_F // _L):
                        g = rows_v[par, r0 + k, pl.ds(fb * _L, _L)] * wk
                        acc_m[fb] = acc_m[fb] + g
                        acc_x[fb] = jnp.maximum(acc_x[fb], g)
                for fb in range(_F // _L):
                    s = self_v[par, v, pl.ds(fb * _L, _L)]
                    out_v[par, v, pl.ds(fb * _L, _L)] = acc_m[fb] * (1.0 / _K) - s
                    out_v[par, v, pl.ds(_F + fb * _L, _L)] = acc_x[fb] - s
                return c2

            lax.fori_loop(0, _C, vert_body, 0)
            pltpu.sync_copy(out_v.at[par], out_hbm.at[pl.ds(base, _C)])

        def pair_body(i, carry):
            c0 = 2 * i
            wait_gather(0)
            wait_in(1)
            start_gather(1)
            compute(0, c0)
            start_in(0, c0 + 2)
            wait_gather(1)
            wait_in(0)
            start_gather(0)
            compute(1, c0 + 1)
            start_in(1, c0 + 3)
            return carry

        with jax.named_scope("work"):
            start_in(0, 0)
            start_in(1, 1)
            wait_in(0)
            start_gather(0)
            lax.fori_loop(0, n_pairs - 1, pair_body, 0)
            c0 = n_chunks - 2
            wait_gather(0)
            wait_in(1)
            start_gather(1)
            compute(0, c0)
            wait_gather(1)
            compute(1, c0 + 1)

    return knn


def kernel(x, neighbor_indices, distancesq, W1, b1, W2, b2):
    v, d = x.shape
    span = _NW * _C * 2
    vp = ((v + span - 1) // span) * span
    pad = vp - v
    xp = jnp.pad(x, ((0, pad), (0, 0)))
    idx_flat = jnp.pad(neighbor_indices, ((0, pad), (0, 0))).reshape(-1)
    dsq_flat = jnp.pad(distancesq, ((0, pad), (0, 0))).reshape(-1)

    knn = _make_knn(vp)
    h1 = _mm_relu(xp, W1, b1, vp)
    out1 = knn(h1, idx_flat, dsq_flat)
    h2 = _mm_relu(out1, W2, b2, vp)
    out2 = knn(h2, idx_flat, dsq_flat)
    return jnp.concatenate([out1[:v], out2[:v], x], axis=-1)


# speedup vs baseline: 2.3517x; 2.3420x over previous
import functools

import jax
import jax.numpy as jnp
from jax import lax
from jax.experimental import pallas as pl
from jax.experimental.pallas import tpu as pltpu
from jax.experimental.pallas import tpu_sc as plsc

_K = 16
_F = 64
_L = 16
_NC = 2
_NS = 16
_NW = _NC * _NS
_C = 32
_CK = _C * _K


def _mm_relu(x, w, b, out_rows, block_rows=512):
    v, d = x.shape
    f = w.shape[1]
    assert out_rows % block_rows == 0
    grid = out_rows // block_rows
    last_blk = (v + block_rows - 1) // block_rows - 1

    def body(x_ref, w_ref, b_ref, o_ref):
        acc = jnp.dot(x_ref[...], w_ref[...], preferred_element_type=jnp.float32)
        o_ref[...] = jnp.maximum(acc + b_ref[...], 0.0)

    return pl.pallas_call(
        body,
        grid=(grid,),
        in_specs=[
            pl.BlockSpec((block_rows, d), lambda i: (jnp.minimum(i, last_blk), 0)),
            pl.BlockSpec((d, f), lambda i: (0, 0)),
            pl.BlockSpec((1, f), lambda i: (0, 0)),
        ],
        out_specs=pl.BlockSpec((block_rows, f), lambda i: (i, 0)),
        out_shape=jax.ShapeDtypeStruct((out_rows, f), jnp.float32),
    )(x, w, b.reshape(1, f))


@functools.lru_cache(maxsize=None)
def _make_knn(vp: int):
    n_w = vp // _NW
    n_chunks = n_w // _C
    n_pairs = n_chunks // 2
    assert n_pairs * 2 * _C == n_w
    mesh = plsc.VectorSubcoreMesh(core_axis_name="c", subcore_axis_name="s")

    @functools.partial(
        pl.kernel,
        out_type=jax.ShapeDtypeStruct((vp, 2 * _F), jnp.float32),
        mesh=mesh,
        compiler_params=pltpu.CompilerParams(use_tc_tiling_on_sc=False),
        scratch_types=[
            pltpu.VMEM((2, _CK), jnp.int32),
            pltpu.VMEM((2, _CK), jnp.float32),
            pltpu.VMEM((2, _CK, _F), jnp.float32),
            pltpu.VMEM((2, _C, _F), jnp.float32),
            pltpu.VMEM((2, _C, 2 * _F), jnp.float32),
            pltpu.SemaphoreType.DMA,
            pltpu.SemaphoreType.DMA,
            pltpu.SemaphoreType.DMA,
            pltpu.SemaphoreType.DMA,
        ],
    )
    def knn(feat_hbm, idx_hbm, dsq_hbm, out_hbm,
            idx_v, dsq_v, rows_v, self_v, out_v,
            sem_in0, sem_in1, sem_g0, sem_g1):
        wid = lax.axis_index("s") * _NC + lax.axis_index("c")
        w_base = wid * n_w
        sem_in = (sem_in0, sem_in1)
        sem_g = (sem_g0, sem_g1)

        def start_in(par, ci):
            base = w_base + ci * _C
            pltpu.async_copy(idx_hbm.at[pl.ds(base * _K, _CK)], idx_v.at[par], sem_in[par])
            pltpu.async_copy(dsq_hbm.at[pl.ds(base * _K, _CK)], dsq_v.at[par], sem_in[par])
            pltpu.async_copy(feat_hbm.at[pl.ds(base, _C)], self_v.at[par], sem_in[par])

        def wait_in(par):
            pltpu.make_async_copy(idx_hbm.at[pl.ds(0, _CK)], idx_v.at[par], sem_in[par]).wait()
            pltpu.make_async_copy(dsq_hbm.at[pl.ds(0, _CK)], dsq_v.at[par], sem_in[par]).wait()
            pltpu.make_async_copy(feat_hbm.at[pl.ds(0, _C)], self_v.at[par], sem_in[par]).wait()

        def start_gather(par):
            pltpu.async_copy(feat_hbm.at[idx_v.at[par]], rows_v.at[par], sem_g[par])

        def wait_gather(par):
            pltpu.make_async_copy(feat_hbm.at[pl.ds(0, _CK)], rows_v.at[par], sem_g[par]).wait()

        def compute(par, ci):
            base = w_base + ci * _C

            def vert_body(v, c2):
                wv = jnp.exp(dsq_v[par, pl.ds(v * _K, _K)] * -10.0)
                r0 = v * _K
                w0 = wv[0]
                acc_m = []
                acc_x = []
                for fb in range(_F // _L):
                    g = rows_v[par, r0, pl.ds(fb * _L, _L)] * w0
                    acc_m.append(g)
                    acc_x.append(g)
                for k in range(1, _K):
                    wk = wv[k]
                    for fb in range(_F // _L):
                        g = rows_v[par, r0 + k, pl.ds(fb * _L, _L)] * wk
                        acc_m[fb] = acc_m[fb] + g
                        acc_x[fb] = jnp.maximum(acc_x[fb], g)
                for fb in range(_F // _L):
                    s = self_v[par, v, pl.ds(fb * _L, _L)]
                    out_v[par, v, pl.ds(fb * _L, _L)] = acc_m[fb] * (1.0 / _K) - s
                    out_v[par, v, pl.ds(_F + fb * _L, _L)] = acc_x[fb] - s
                return c2

            lax.fori_loop(0, _C, vert_body, 0)
            pltpu.sync_copy(out_v.at[par], out_hbm.at[pl.ds(base, _C)])

        def pair_body(i, carry):
            c0 = 2 * i
            wait_gather(0)
            wait_in(1)
            start_gather(1)
            compute(0, c0)
            start_in(0, c0 + 2)
            wait_gather(1)
            wait_in(0)
            start_gather(0)
            compute(1, c0 + 1)
            start_in(1, c0 + 3)
            return carry

        with jax.named_scope("work"):
            start_in(0, 0)
            start_in(1, 1)
            wait_in(0)
            start_gather(0)
            lax.fori_loop(0, n_pairs - 1, pair_body, 0)
            c0 = n_chunks - 2
            wait_gather(0)
            wait_in(1)
            start_gather(1)
            compute(0, c0)
            wait_gather(1)
            compute(1, c0 + 1)

    return knn


def kernel(x, neighbor_indices, distancesq, W1, b1, W2, b2):
    v, d = x.shape
    span = _NW * _C * 2
    vp = ((v + span - 1) // span) * span
    pad = vp - v
    xp = jnp.pad(x, ((0, pad), (0, 0)))
    pad_idx = (jnp.arange(pad * _K, dtype=jnp.int32) * 9973) % v
    idx_flat = jnp.concatenate(
        [neighbor_indices.reshape(-1), pad_idx])
    dsq_flat = jnp.pad(distancesq, ((0, pad), (0, 0))).reshape(-1)

    knn = _make_knn(vp)
    h1 = _mm_relu(xp, W1, b1, vp)
    out1 = knn(h1, idx_flat, dsq_flat)
    h2 = _mm_relu(out1, W2, b2, vp)
    out2 = knn(h2, idx_flat, dsq_flat)
    return jnp.concatenate([out1[:v], out2[:v], x], axis=-1)


# speedup vs baseline: 3.0045x; 1.2776x over previous
import functools

import jax
import jax.numpy as jnp
from jax import lax
from jax.experimental import pallas as pl
from jax.experimental.pallas import tpu as pltpu
from jax.experimental.pallas import tpu_sc as plsc

_K = 16
_F = 64
_L = 16
_NC = 2
_NS = 16
_NW = _NC * _NS
_C = 28
_CK = _C * _K


def _mm_relu(x, w, b, block_rows=2048):
    v, d = x.shape
    f = w.shape[1]
    grid = (v + block_rows - 1) // block_rows

    def body(x_ref, w_ref, b_ref, o_ref):
        acc = jnp.dot(x_ref[...], w_ref[...], preferred_element_type=jnp.float32)
        o_ref[...] = jnp.maximum(acc + b_ref[...], 0.0)

    return pl.pallas_call(
        body,
        grid=(grid,),
        in_specs=[
            pl.BlockSpec((block_rows, d), lambda i: (i, 0)),
            pl.BlockSpec((d, f), lambda i: (0, 0)),
            pl.BlockSpec((1, f), lambda i: (0, 0)),
        ],
        out_specs=pl.BlockSpec((block_rows, f), lambda i: (i, 0)),
        out_shape=jax.ShapeDtypeStruct((v, f), jnp.float32),
    )(x, w, b.reshape(1, f))


@functools.lru_cache(maxsize=None)
def _make_knn(v: int):
    span = 2 * _C
    n_wv = ((v + _NW - 1) // _NW + span - 1) // span * span
    n_chunks = n_wv // _C
    n_pairs = n_chunks // 2
    last_base = v - _C
    mesh = plsc.VectorSubcoreMesh(core_axis_name="c", subcore_axis_name="s")

    @functools.partial(
        pl.kernel,
        out_type=jax.ShapeDtypeStruct((v, 2 * _F), jnp.float32),
        mesh=mesh,
        compiler_params=pltpu.CompilerParams(use_tc_tiling_on_sc=False),
        scratch_types=[
            pltpu.VMEM((2, _CK), jnp.int32),
            pltpu.VMEM((2, _CK), jnp.float32),
            pltpu.VMEM((2, _CK, _F), jnp.float32),
            pltpu.VMEM((2, _C, _F), jnp.float32),
            pltpu.VMEM((2, _C, 2 * _F), jnp.float32),
            pltpu.SemaphoreType.DMA,
            pltpu.SemaphoreType.DMA,
            pltpu.SemaphoreType.DMA,
            pltpu.SemaphoreType.DMA,
        ],
    )
    def knn(feat_hbm, idx_hbm, dsq_hbm, out_hbm,
            idx_v, dsq_v, rows_v, self_v, out_v,
            sem_in0, sem_in1, sem_g0, sem_g1):
        wid = lax.axis_index("s") * _NC + lax.axis_index("c")
        w_base = wid * n_wv
        sem_in = (sem_in0, sem_in1)
        sem_g = (sem_g0, sem_g1)

        def chunk_base(ci):
            return jnp.minimum(w_base + ci * _C, last_base)

        def start_in(par, ci):
            base = chunk_base(ci)
            pltpu.async_copy(idx_hbm.at[pl.ds(base * _K, _CK)], idx_v.at[par], sem_in[par])
            pltpu.async_copy(dsq_hbm.at[pl.ds(base * _K, _CK)], dsq_v.at[par], sem_in[par])
            pltpu.async_copy(feat_hbm.at[pl.ds(base, _C)], self_v.at[par], sem_in[par])

        def wait_in(par):
            pltpu.make_async_copy(idx_hbm.at[pl.ds(0, _CK)], idx_v.at[par], sem_in[par]).wait()
            pltpu.make_async_copy(dsq_hbm.at[pl.ds(0, _CK)], dsq_v.at[par], sem_in[par]).wait()
            pltpu.make_async_copy(feat_hbm.at[pl.ds(0, _C)], self_v.at[par], sem_in[par]).wait()

        def start_gather(par):
            pltpu.async_copy(feat_hbm.at[idx_v.at[par]], rows_v.at[par], sem_g[par])

        def wait_gather(par):
            pltpu.make_async_copy(feat_hbm.at[pl.ds(0, _CK)], rows_v.at[par], sem_g[par]).wait()

        def compute(par, ci):
            base = chunk_base(ci)

            def vert_body(vi, c2):
                wv = jnp.exp(dsq_v[par, pl.ds(vi * _K, _K)] * -10.0)
                r0 = vi * _K
                w0 = wv[0]
                acc_m = []
                acc_x = []
                for fb in range(_F // _L):
                    g = rows_v[par, r0, pl.ds(fb * _L, _L)] * w0
                    acc_m.append(g)
                    acc_x.append(g)
                for k in range(1, _K):
                    wk = wv[k]
                    for fb in range(_F // _L):
                        g = rows_v[par, r0 + k, pl.ds(fb * _L, _L)] * wk
                        acc_m[fb] = acc_m[fb] + g
                        acc_x[fb] = jnp.maximum(acc_x[fb], g)
                for fb in range(_F // _L):
                    s = self_v[par, vi, pl.ds(fb * _L, _L)]
                    out_v[par, vi, pl.ds(fb * _L, _L)] = acc_m[fb] * (1.0 / _K) - s
                    out_v[par, vi, pl.ds(_F + fb * _L, _L)] = acc_x[fb] - s
                return c2

            lax.fori_loop(0, _C, vert_body, 0)
            pltpu.sync_copy(out_v.at[par], out_hbm.at[pl.ds(base, _C)])

        def pair_body(i, carry):
            c0 = 2 * i
            wait_gather(0)
            wait_in(1)
            start_gather(1)
            compute(0, c0)
            start_in(0, c0 + 2)
            wait_gather(1)
            wait_in(0)
            start_gather(0)
            compute(1, c0 + 1)
            start_in(1, c0 + 3)
            return carry

        start_in(0, 0)
        start_in(1, 1)
        wait_in(0)
        start_gather(0)
        lax.fori_loop(0, n_pairs - 1, pair_body, 0)
        c0 = n_chunks - 2
        wait_gather(0)
        wait_in(1)
        start_gather(1)
        compute(0, c0)
        wait_gather(1)
        compute(1, c0 + 1)

    return knn


def kernel(x, neighbor_indices, distancesq, W1, b1, W2, b2):
    v, d = x.shape
    idx_flat = neighbor_indices.reshape(-1)
    dsq_flat = distancesq.reshape(-1)

    knn = _make_knn(v)
    h1 = _mm_relu(x, W1, b1)
    out1 = knn(h1, idx_flat, dsq_flat)
    h2 = _mm_relu(out1, W2, b2)
    out2 = knn(h2, idx_flat, dsq_flat)
    return jnp.concatenate([out1, out2, x], axis=-1)


# speedup vs baseline: 3.0211x; 1.0055x over previous
import functools

import jax
import jax.numpy as jnp
from jax import lax
from jax.experimental import pallas as pl
from jax.experimental.pallas import tpu as pltpu
from jax.experimental.pallas import tpu_sc as plsc

_K = 16
_F = 64
_L = 16
_NC = 2
_NS = 16
_NW = _NC * _NS
_C = 28
_CK = _C * _K


def _mm_relu(x, w, b, block_rows=2048):
    v, d = x.shape
    f = w.shape[1]
    grid = (v + block_rows - 1) // block_rows

    def body(x_ref, w_ref, b_ref, o_ref):
        acc = jnp.dot(x_ref[...], w_ref[...], preferred_element_type=jnp.float32)
        o_ref[...] = jnp.maximum(acc + b_ref[...], 0.0)

    return pl.pallas_call(
        body,
        grid=(grid,),
        in_specs=[
            pl.BlockSpec((block_rows, d), lambda i: (i, 0)),
            pl.BlockSpec((d, f), lambda i: (0, 0)),
            pl.BlockSpec((1, f), lambda i: (0, 0)),
        ],
        out_specs=pl.BlockSpec((block_rows, f), lambda i: (i, 0)),
        out_shape=jax.ShapeDtypeStruct((v, f), jnp.float32),
    )(x, w, b.reshape(1, f))


@functools.lru_cache(maxsize=None)
def _make_knn(v: int):
    span = 2 * _C
    n_wv = ((v + _NW - 1) // _NW + span - 1) // span * span
    n_chunks = n_wv // _C
    n_pairs = n_chunks // 2
    last_base = v - _C
    mesh = plsc.VectorSubcoreMesh(core_axis_name="c", subcore_axis_name="s")

    @functools.partial(
        pl.kernel,
        out_type=jax.ShapeDtypeStruct((v, 2 * _F), jnp.float32),
        mesh=mesh,
        compiler_params=pltpu.CompilerParams(use_tc_tiling_on_sc=False),
        scratch_types=[
            pltpu.VMEM((2, _CK), jnp.int32),
            pltpu.VMEM((2, _CK), jnp.float32),
            pltpu.VMEM((2, _CK, _F), jnp.float32),
            pltpu.VMEM((2, _C, _F), jnp.float32),
            pltpu.VMEM((2, _C, 2 * _F), jnp.float32),
            pltpu.SemaphoreType.DMA,
            pltpu.SemaphoreType.DMA,
            pltpu.SemaphoreType.DMA,
            pltpu.SemaphoreType.DMA,
        ],
    )
    def knn(feat_hbm, idx_hbm, dsq_hbm, out_hbm,
            idx_v, dsq_v, rows_v, self_v, out_v,
            sem_in0, sem_in1, sem_g0, sem_g1):
        wid = lax.axis_index("s") * _NC + lax.axis_index("c")
        w_base = wid * n_wv
        sem_in = (sem_in0, sem_in1)
        sem_g = (sem_g0, sem_g1)

        def chunk_base(ci):
            return jnp.minimum(w_base + ci * _C, last_base)

        def start_in(par, ci):
            base = chunk_base(ci)
            pltpu.async_copy(idx_hbm.at[pl.ds(base * _K, _CK)], idx_v.at[par], sem_in[par])
            pltpu.async_copy(dsq_hbm.at[pl.ds(base * _K, _CK)], dsq_v.at[par], sem_in[par])
            pltpu.async_copy(feat_hbm.at[pl.ds(base, _C)], self_v.at[par], sem_in[par])

        def wait_in(par):
            pltpu.make_async_copy(idx_hbm.at[pl.ds(0, _CK)], idx_v.at[par], sem_in[par]).wait()
            pltpu.make_async_copy(dsq_hbm.at[pl.ds(0, _CK)], dsq_v.at[par], sem_in[par]).wait()
            pltpu.make_async_copy(feat_hbm.at[pl.ds(0, _C)], self_v.at[par], sem_in[par]).wait()

        def start_gather(par):
            pltpu.async_copy(feat_hbm.at[idx_v.at[par]], rows_v.at[par], sem_g[par])

        def wait_gather(par):
            pltpu.make_async_copy(feat_hbm.at[pl.ds(0, _CK)], rows_v.at[par], sem_g[par]).wait()

        def compute(par, ci):
            base = chunk_base(ci)

            def vert_body(vi, c2):
                wv = jnp.exp(dsq_v[par, pl.ds(vi * _K, _K)] * -10.0)
                r0 = vi * _K
                w0 = wv[0]
                acc_m = []
                acc_x = []
                for fb in range(_F // _L):
                    g = rows_v[par, r0, pl.ds(fb * _L, _L)] * w0
                    acc_m.append(g)
                    acc_x.append(g)
                for k in range(1, _K):
                    wk = wv[k]
                    for fb in range(_F // _L):
                        g = rows_v[par, r0 + k, pl.ds(fb * _L, _L)] * wk
                        acc_m[fb] = acc_m[fb] + g
                        acc_x[fb] = jnp.maximum(acc_x[fb], g)
                for fb in range(_F // _L):
                    s = self_v[par, vi, pl.ds(fb * _L, _L)]
                    out_v[par, vi, pl.ds(fb * _L, _L)] = acc_m[fb] * (1.0 / _K) - s
                    out_v[par, vi, pl.ds(_F + fb * _L, _L)] = acc_x[fb] - s
                return c2

            lax.fori_loop(0, _C, vert_body, 0, unroll=2)
            pltpu.sync_copy(out_v.at[par], out_hbm.at[pl.ds(base, _C)])

        def pair_body(i, carry):
            c0 = 2 * i
            wait_gather(0)
            wait_in(1)
            start_gather(1)
            compute(0, c0)
            start_in(0, c0 + 2)
            wait_gather(1)
            wait_in(0)
            start_gather(0)
            compute(1, c0 + 1)
            start_in(1, c0 + 3)
            return carry

        start_in(0, 0)
        start_in(1, 1)
        wait_in(0)
        start_gather(0)
        lax.fori_loop(0, n_pairs - 1, pair_body, 0)
        c0 = n_chunks - 2
        wait_gather(0)
        wait_in(1)
        start_gather(1)
        compute(0, c0)
        wait_gather(1)
        compute(1, c0 + 1)

    return knn


def kernel(x, neighbor_indices, distancesq, W1, b1, W2, b2):
    v, d = x.shape
    idx_flat = neighbor_indices.reshape(-1)
    dsq_flat = distancesq.reshape(-1)

    knn = _make_knn(v)
    h1 = _mm_relu(x, W1, b1)
    out1 = knn(h1, idx_flat, dsq_flat)
    h2 = _mm_relu(out1, W2, b2)
    out2 = knn(h2, idx_flat, dsq_flat)
    return jnp.concatenate([out1, out2, x], axis=-1)


# speedup vs baseline: 3.1430x; 1.0404x over previous
import functools

import jax
import jax.numpy as jnp
from jax import lax
from jax.experimental import pallas as pl
from jax.experimental.pallas import tpu as pltpu
from jax.experimental.pallas import tpu_sc as plsc

_K = 16
_F = 64
_L = 16
_NC = 2
_NS = 16
_NW = _NC * _NS
_C = 28
_CK = _C * _K


def _mm_relu(x, w, b, block_rows=2048):
    v, d = x.shape
    f = w.shape[1]
    grid = (v + block_rows - 1) // block_rows

    def body(x_ref, w_ref, b_ref, o_ref):
        acc = jnp.dot(x_ref[...], w_ref[...], preferred_element_type=jnp.float32)
        o_ref[...] = jnp.maximum(acc + b_ref[...], 0.0)

    return pl.pallas_call(
        body,
        grid=(grid,),
        in_specs=[
            pl.BlockSpec((block_rows, d), lambda i: (i, 0)),
            pl.BlockSpec((d, f), lambda i: (0, 0)),
            pl.BlockSpec((1, f), lambda i: (0, 0)),
        ],
        out_specs=pl.BlockSpec((block_rows, f), lambda i: (i, 0)),
        out_shape=jax.ShapeDtypeStruct((v, f), jnp.float32),
    )(x, w, b.reshape(1, f))


@functools.lru_cache(maxsize=None)
def _make_knn(v: int):
    span = 2 * _C
    n_wv = ((v + _NW - 1) // _NW + span - 1) // span * span
    n_chunks = n_wv // _C
    n_pairs = n_chunks // 2
    last_base = v - _C
    mesh = plsc.VectorSubcoreMesh(core_axis_name="c", subcore_axis_name="s")

    @functools.partial(
        pl.kernel,
        out_type=jax.ShapeDtypeStruct((v, 2 * _F), jnp.float32),
        mesh=mesh,
        compiler_params=pltpu.CompilerParams(use_tc_tiling_on_sc=False),
        scratch_types=[
            pltpu.VMEM((2, _CK), jnp.int32),
            pltpu.VMEM((2, _CK), jnp.float32),
            pltpu.VMEM((2, _CK, _F), jnp.float32),
            pltpu.VMEM((2, _C, _F), jnp.float32),
            pltpu.VMEM((2, _C, 2 * _F), jnp.float32),
            pltpu.SemaphoreType.DMA,
            pltpu.SemaphoreType.DMA,
            pltpu.SemaphoreType.DMA,
            pltpu.SemaphoreType.DMA,
            pltpu.SemaphoreType.DMA,
            pltpu.SemaphoreType.DMA,
        ],
    )
    def knn(feat_hbm, idx_hbm, dsq_hbm, out_hbm,
            idx_v, dsq_v, rows_v, self_v, out_v,
            sem_in0, sem_in1, sem_g0, sem_g1, sem_o0, sem_o1):
        wid = lax.axis_index("s") * _NC + lax.axis_index("c")
        w_base = wid * n_wv
        sem_in = (sem_in0, sem_in1)
        sem_g = (sem_g0, sem_g1)
        sem_o = (sem_o0, sem_o1)

        def chunk_base(ci):
            return jnp.minimum(w_base + ci * _C, last_base)

        def start_in(par, ci):
            base = chunk_base(ci)
            pltpu.async_copy(idx_hbm.at[pl.ds(base * _K, _CK)], idx_v.at[par], sem_in[par])
            pltpu.async_copy(dsq_hbm.at[pl.ds(base * _K, _CK)], dsq_v.at[par], sem_in[par])
            pltpu.async_copy(feat_hbm.at[pl.ds(base, _C)], self_v.at[par], sem_in[par])

        def wait_in(par):
            pltpu.make_async_copy(idx_hbm.at[pl.ds(0, _CK)], idx_v.at[par], sem_in[par]).wait()
            pltpu.make_async_copy(dsq_hbm.at[pl.ds(0, _CK)], dsq_v.at[par], sem_in[par]).wait()
            pltpu.make_async_copy(feat_hbm.at[pl.ds(0, _C)], self_v.at[par], sem_in[par]).wait()

        def start_gather(par):
            pltpu.async_copy(feat_hbm.at[idx_v.at[par]], rows_v.at[par], sem_g[par])

        def wait_gather(par):
            pltpu.make_async_copy(feat_hbm.at[pl.ds(0, _CK)], rows_v.at[par], sem_g[par]).wait()

        def wait_out(par):
            pltpu.make_async_copy(out_v.at[par], out_hbm.at[pl.ds(0, _C)], sem_o[par]).wait()

        def compute(par, ci):
            base = chunk_base(ci)

            @pl.when(ci >= 2)
            def _():
                wait_out(par)

            def vert_body(vi, c2):
                wv = jnp.exp(dsq_v[par, pl.ds(vi * _K, _K)] * -10.0)
                r0 = vi * _K
                w0 = wv[0]
                acc_m = []
                acc_x = []
                for fb in range(_F // _L):
                    g = rows_v[par, r0, pl.ds(fb * _L, _L)] * w0
                    acc_m.append(g)
                    acc_x.append(g)
                for k in range(1, _K):
                    wk = wv[k]
                    for fb in range(_F // _L):
                        g = rows_v[par, r0 + k, pl.ds(fb * _L, _L)] * wk
                        acc_m[fb] = acc_m[fb] + g
                        acc_x[fb] = jnp.maximum(acc_x[fb], g)
                for fb in range(_F // _L):
                    s = self_v[par, vi, pl.ds(fb * _L, _L)]
                    out_v[par, vi, pl.ds(fb * _L, _L)] = acc_m[fb] * (1.0 / _K) - s
                    out_v[par, vi, pl.ds(_F + fb * _L, _L)] = acc_x[fb] - s
                return c2

            lax.fori_loop(0, _C, vert_body, 0, unroll=2)
            pltpu.async_copy(out_v.at[par], out_hbm.at[pl.ds(base, _C)], sem_o[par])

        def pair_body(i, carry):
            c0 = 2 * i
            wait_gather(0)
            wait_in(1)
            start_gather(1)
            compute(0, c0)
            start_in(0, c0 + 2)
            wait_gather(1)
            wait_in(0)
            start_gather(0)
            compute(1, c0 + 1)
            start_in(1, c0 + 3)
            return carry

        start_in(0, 0)
        start_in(1, 1)
        wait_in(0)
        start_gather(0)
        lax.fori_loop(0, n_pairs - 1, pair_body, 0)
        c0 = n_chunks - 2
        wait_gather(0)
        wait_in(1)
        start_gather(1)
        compute(0, jnp.int32(c0))
        wait_gather(1)
        compute(1, jnp.int32(c0 + 1))
        wait_out(0)
        wait_out(1)

    return knn


def kernel(x, neighbor_indices, distancesq, W1, b1, W2, b2):
    v, d = x.shape
    idx_flat = neighbor_indices.reshape(-1)
    dsq_flat = distancesq.reshape(-1)

    knn = _make_knn(v)
    h1 = _mm_relu(x, W1, b1)
    out1 = knn(h1, idx_flat, dsq_flat)
    h2 = _mm_relu(out1, W2, b2)
    out2 = knn(h2, idx_flat, dsq_flat)
    return jnp.concatenate([out1, out2, x], axis=-1)
